# Initial kernel scaffold; baseline (speedup 1.0000x reference)
import jax
import jax.numpy as jnp
from jax.experimental import pallas as pl


def kernel(x, edge_index, edge_attr, conditions, batch, ne_W1, ne_b1, ne_W2, ne_b2, ee_W1, ee_b1, ee_W2, ee_b2, ce_W1, ce_b1, ce_W2, ce_b2, a_Wq, a_bq, a_Wk, a_bk, a_Wv, a_bv, a_Wo, a_bo, pe_W1, pe_b1, pe_W2, pe_b2, pn_W1, pn_b1, pn_W2, pn_b2, dec_W1, dec_b1, dec_W2, dec_b2):
    raise NotImplementedError("write your pallas kernel here")


# speedup vs baseline: 2.3852x; 2.3852x over previous
import functools

import jax
import jax.numpy as jnp
from jax import lax
from jax.experimental import pallas as pl
from jax.experimental.pallas import tpu as pltpu
from jax.experimental.pallas import tpu_sc as plsc

N = 10000
E = 320000
H = 64
B = 8
L = 3

NC, NS = 2, 16
NW = NC * NS
EPW = E // NW
CH = 80
NCHUNK = EPW // CH
NPT = N // NS

NBLK = 2000
EBLK = 8000


def _sc_mesh():
    return plsc.VectorSubcoreMesh(
        core_axis_name="c", subcore_axis_name="s",
        num_cores=NC, num_subcores=NS)


def _sc_gather_body(ap_hbm, bv_hbm, row_hbm, col_hbm, g_hbm,
                    ridx, cidx, gbuf, sem1, sem2):
    c = lax.axis_index("c")
    s = lax.axis_index("s")
    wid = s * NC + c
    base0 = wid * EPW
    pltpu.sync_copy(row_hbm.at[pl.ds(base0, EPW)], ridx)
    pltpu.sync_copy(col_hbm.at[pl.ds(base0, EPW)], cidx)

    def body(i, carry):
        off = i * CH
        pltpu.async_copy(ap_hbm.at[ridx.at[pl.ds(off, CH)]], gbuf, sem1).wait()
        pltpu.async_copy(bv_hbm.at[cidx.at[pl.ds(off, CH)]], gbuf, sem2,
                         add=True).wait()
        pltpu.sync_copy(gbuf, g_hbm.at[pl.ds(base0 + off, CH)])
        return carry

    lax.fori_loop(0, NCHUNK, body, 0)


@functools.cache
def _sc_gather_kernel():
    return pl.kernel(
        _sc_gather_body,
        out_type=jax.ShapeDtypeStruct((E, 2 * H), jnp.float32),
        mesh=_sc_mesh(),
        scratch_types=[
            pltpu.VMEM((EPW,), jnp.int32),
            pltpu.VMEM((EPW,), jnp.int32),
            pltpu.VMEM((CH, 2 * H), jnp.float32),
            pltpu.SemaphoreType.DMA,
            pltpu.SemaphoreType.DMA,
        ],
    )


def _sc_gather(ap, bv, row, col):
    return _sc_gather_kernel()(ap, bv, row, col)


_DR = 16
_NDCH = N // _DR


def _sc_scatter_body(ea_hbm, row_hbm, out_hbm, ridx, ebuf, dbuf, shared):
    c = lax.axis_index("c")
    s = lax.axis_index("s")
    zero = jnp.zeros((16,), jnp.float32)

    def zfill(i, carry):
        for k in range(2 * H // 16):
            dbuf[i, pl.ds(k * 16, 16)] = zero
        return carry

    lax.fori_loop(0, _DR, zfill, 0)
    nd = 39 + jnp.where(s == 0, 1, 0)

    def zbody(i, carry):
        pltpu.sync_copy(dbuf, shared.at[pl.ds((s + i * NS) * _DR, _DR)])
        return carry

    lax.fori_loop(0, nd, zbody, 0)
    plsc.subcore_barrier()

    base0 = (c * NS + s) * EPW

    def body(i, carry):
        off = base0 + i * CH
        pltpu.sync_copy(row_hbm.at[pl.ds(off, CH)], ridx.at[0])
        pltpu.sync_copy(ea_hbm.at[pl.ds(off, CH)], ebuf)
        pltpu.sync_copy(ebuf, shared.at[ridx.at[0]], add=True)
        return carry

    lax.fori_loop(0, NCHUNK, body, 0)
    plsc.subcore_barrier()

    def dbody(i, carry):
        rbase = (s + i * NS) * _DR
        pltpu.sync_copy(shared.at[pl.ds(rbase, _DR)], dbuf)
        pltpu.sync_copy(dbuf, out_hbm.at[pl.ds(c * N + rbase, _DR)])
        return carry

    lax.fori_loop(0, nd, dbody, 0)


@functools.cache
def _sc_scatter_kernel():
    return pl.kernel(
        _sc_scatter_body,
        out_type=jax.ShapeDtypeStruct((NC * N, 2 * H), jnp.float32),
        mesh=_sc_mesh(),
        scratch_types=[
            pltpu.VMEM((1, CH), jnp.int32),
            pltpu.VMEM((CH, 2 * H), jnp.float32),
            pltpu.VMEM((_DR, 2 * H), jnp.float32),
            pltpu.VMEM_SHARED((N, 2 * H), jnp.float32),
        ],
    )


def _sc_scatter(ea, row):
    return _sc_scatter_kernel()(ea, row)


def _full(shape):
    nd = len(shape)
    return pl.BlockSpec(shape, lambda i, _nd=nd: (0,) * _nd)


def _rows(blk, width):
    return pl.BlockSpec((blk, width), lambda i: (i, 0))


def _mm(a, b):
    return jnp.dot(a, b, precision=lax.Precision.HIGHEST)


def _graph_body(cond, ceW1, ceb1, ceW2, ceb2, Wv, bv, Wo, bo,
                Wu, peb1, Watt, pnb1, u_out, att_out):
    u = _mm(jnp.maximum(_mm(cond[...], ceW1[...]) + ceb1[...], 0.0),
            ceW2[...]) + ceb2[...]
    att = _mm(_mm(u, Wv[...]) + bv[...], Wo[...]) + bo[...]
    for l in range(L):
        u_out[l] = _mm(u, Wu[l]) + peb1[l]
        att_out[l] = _mm(att, Watt[l]) + pnb1[l]


def _enc_node_body(x, bat, neW1, neb1, neW2, neb2, Wr, Wc, U0,
                   h_out, a_out, b_out):
    h = _mm(jnp.maximum(_mm(x[...], neW1[...]) + neb1[...], 0.0),
            neW2[...]) + neb2[...]
    oh = (bat[...] == lax.broadcasted_iota(jnp.int32, (NBLK, B), 1)
          ).astype(jnp.float32)
    z = jnp.zeros((NBLK, H), jnp.float32)
    h_out[...] = h
    a_out[...] = jnp.concatenate([_mm(h, Wr[...]) + _mm(oh, U0[...]), z],
                                 axis=1)
    b_out[...] = jnp.concatenate([z, _mm(h, Wc[...])], axis=1)


def _enc_edge_body(eattr, W1, b1, W2, b2, out):
    ea = _mm(jnp.maximum(_mm(eattr[...], W1[...]) + b1[...], 0.0),
             W2[...]) + b2[...]
    out[...] = jnp.concatenate([ea, jnp.zeros((EBLK, H), jnp.float32)], axis=1)


def _edge_mlp_body(g, ea, We, W2, b2, out):
    gv = g[...]
    hid = jnp.maximum(gv[:, :H] + gv[:, H:] + _mm(ea[:, :H], We[...]), 0.0)
    eo = _mm(hid, W2[...]) + b2[...]
    out[...] = jnp.concatenate([eo, jnp.zeros((EBLK, H), jnp.float32)], axis=1)


def _node_body(h, agg0, agg1, bat, Wh, Wagg, ATTl, W2, b2, Wr, Wc, Un,
               h_out, a_out, b_out):
    hv = h[...]
    oh = (bat[...] == lax.broadcasted_iota(jnp.int32, (NBLK, B), 1)
          ).astype(jnp.float32)
    pre = (_mm(hv, Wh[...]) + _mm(agg0[:, :H] + agg1[:, :H], Wagg[...])
           + _mm(oh, ATTl[...]))
    hn = _mm(jnp.maximum(pre, 0.0), W2[...]) + b2[...] + hv
    z = jnp.zeros((NBLK, H), jnp.float32)
    h_out[...] = hn
    a_out[...] = jnp.concatenate([_mm(hn, Wr[...]) + _mm(oh, Un[...]), z],
                                 axis=1)
    b_out[...] = jnp.concatenate([z, _mm(hn, Wc[...])], axis=1)


def _dec_body(h, W1, b1, W2, b2, out):
    out[...] = _mm(jnp.maximum(_mm(h[...], W1[...]) + b1[...], 0.0),
                   W2[...]) + b2[...]


def kernel(x, edge_index, edge_attr, conditions, batch,
           ne_W1, ne_b1, ne_W2, ne_b2, ee_W1, ee_b1, ee_W2, ee_b2,
           ce_W1, ce_b1, ce_W2, ce_b2,
           a_Wq, a_bq, a_Wk, a_bk, a_Wv, a_bv, a_Wo, a_bo,
           pe_W1, pe_b1, pe_W2, pe_b2, pn_W1, pn_b1, pn_W2, pn_b2,
           dec_W1, dec_b1, dec_W2, dec_b2):
    f32 = jnp.float32
    row = edge_index[0]
    col = edge_index[1]
    bat2 = batch.reshape(N, 1)

    Wr = pe_W1[:, 0:H, :]
    Wc = pe_W1[:, H:2 * H, :]
    We = pe_W1[:, 2 * H:3 * H, :]
    Wu = pe_W1[:, 3 * H:4 * H, :]
    Wh = pn_W1[:, 0:H, :]
    Wagg = pn_W1[:, H:2 * H, :]
    Watt = pn_W1[:, 2 * H:3 * H, :]

    r1 = lambda b: b.reshape(1, -1)
    r2 = lambda b: b.reshape(L, 1, -1)

    U, ATT = pl.pallas_call(
        _graph_body,
        out_shape=[jax.ShapeDtypeStruct((L, B, H), f32),
                   jax.ShapeDtypeStruct((L, B, H), f32)],
    )(conditions, ce_W1, r1(ce_b1), ce_W2, r1(ce_b2),
      a_Wv, r1(a_bv), a_Wo, r1(a_bo), Wu, r2(pe_b1), Watt, r2(pn_b1))

    gn = N // NBLK
    h, Ap, Bv = pl.pallas_call(
        _enc_node_body,
        grid=(gn,),
        in_specs=[_rows(NBLK, 128), _rows(NBLK, 1),
                  _full((128, H)), _full((1, H)), _full((H, H)), _full((1, H)),
                  _full((H, H)), _full((H, H)), _full((B, H))],
        out_specs=[_rows(NBLK, H), _rows(NBLK, 2 * H), _rows(NBLK, 2 * H)],
        out_shape=[jax.ShapeDtypeStruct((N, H), f32),
                   jax.ShapeDtypeStruct((N, 2 * H), f32),
                   jax.ShapeDtypeStruct((N, 2 * H), f32)],
    )(x, bat2, ne_W1, r1(ne_b1), ne_W2, r1(ne_b2), Wr[0], Wc[0], U[0])

    ge = E // EBLK
    ea = pl.pallas_call(
        _enc_edge_body,
        grid=(ge,),
        in_specs=[_rows(EBLK, 16), _full((16, H)), _full((1, H)),
                  _full((H, H)), _full((1, H))],
        out_specs=_rows(EBLK, 2 * H),
        out_shape=jax.ShapeDtypeStruct((E, 2 * H), f32),
    )(edge_attr, ee_W1, r1(ee_b1), ee_W2, r1(ee_b2))

    for l in range(L):
        G = _sc_gather(Ap, Bv, row, col)
        ea = pl.pallas_call(
            _edge_mlp_body,
            grid=(ge,),
            in_specs=[_rows(EBLK, 2 * H), _rows(EBLK, 2 * H), _full((H, H)),
                      _full((H, H)), _full((1, H))],
            out_specs=_rows(EBLK, 2 * H),
            out_shape=jax.ShapeDtypeStruct((E, 2 * H), f32),
        )(G, ea, We[l], pe_W2[l], r1(pe_b2[l]))

        aggs = _sc_scatter(ea, row)
        ln = (l + 1) % L
        h, Ap, Bv = pl.pallas_call(
            _node_body,
            grid=(gn,),
            in_specs=[_rows(NBLK, H), _rows(NBLK, 2 * H), _rows(NBLK, 2 * H),
                      _rows(NBLK, 1)] +
                     [_full((H, H))] * 2 + [_full((B, H)), _full((H, H)),
                      _full((1, H)), _full((H, H)), _full((H, H)),
                      _full((B, H))],
            out_specs=[_rows(NBLK, H), _rows(NBLK, 2 * H), _rows(NBLK, 2 * H)],
            out_shape=[jax.ShapeDtypeStruct((N, H), f32),
                       jax.ShapeDtypeStruct((N, 2 * H), f32),
                       jax.ShapeDtypeStruct((N, 2 * H), f32)],
        )(h, aggs[:N], aggs[N:], bat2, Wh[l], Wagg[l], ATT[l],
          pn_W2[l], r1(pn_b2[l]), Wr[ln], Wc[ln], U[ln])

    out = pl.pallas_call(
        _dec_body,
        grid=(gn,),
        in_specs=[_rows(NBLK, H), _full((H, H)), _full((1, H)),
                  _full((H, 128)), _full((1, 128))],
        out_specs=_rows(NBLK, 128),
        out_shape=jax.ShapeDtypeStruct((N, 128), f32),
    )(h, dec_W1, r1(dec_b1), dec_W2, r1(dec_b2))
    return out


# speedup vs baseline: 2.8492x; 1.1945x over previous
import functools

import jax
import jax.numpy as jnp
from jax import lax
from jax.experimental import pallas as pl
from jax.experimental.pallas import tpu as pltpu
from jax.experimental.pallas import tpu_sc as plsc

N = 10000
E = 320000
H = 64
B = 8
L = 3

NC, NS = 2, 16
NW = NC * NS
EPW = E // NW
CH = 200
NCHUNK = EPW // CH
NPT = N // NS

NBLK = 2000
EBLK = 8000


def _sc_mesh():
    return plsc.VectorSubcoreMesh(
        core_axis_name="c", subcore_axis_name="s",
        num_cores=NC, num_subcores=NS)


def _sc_gather_body(ap_hbm, bv_hbm, row_hbm, col_hbm, g_hbm,
                    ridx, cidx, gb0, gb1, sa0, sa1, sb0, sb1, so0, so1):
    c = lax.axis_index("c")
    s = lax.axis_index("s")
    wid = s * NC + c
    base0 = wid * EPW
    pltpu.sync_copy(row_hbm.at[pl.ds(base0, EPW)], ridx)
    pltpu.sync_copy(col_hbm.at[pl.ds(base0, EPW)], cidx)

    def body(i, carry):
        o0 = (2 * i) * CH
        o1 = (2 * i + 1) * CH
        d0 = pltpu.async_copy(ap_hbm.at[ridx.at[pl.ds(o0, CH)]], gb0, sa0)
        d1 = pltpu.async_copy(ap_hbm.at[ridx.at[pl.ds(o1, CH)]], gb1, sa1)
        d0.wait()
        e0 = pltpu.async_copy(bv_hbm.at[cidx.at[pl.ds(o0, CH)]], gb0, sb0,
                              add=True)
        d1.wait()
        e1 = pltpu.async_copy(bv_hbm.at[cidx.at[pl.ds(o1, CH)]], gb1, sb1,
                              add=True)
        e0.wait()
        f0 = pltpu.async_copy(gb0, g_hbm.at[pl.ds(base0 + o0, CH)], so0)
        e1.wait()
        f1 = pltpu.async_copy(gb1, g_hbm.at[pl.ds(base0 + o1, CH)], so1)
        f0.wait()
        f1.wait()
        return carry

    lax.fori_loop(0, NCHUNK // 2, body, 0)


@functools.cache
def _sc_gather_kernel():
    return pl.kernel(
        _sc_gather_body,
        out_type=jax.ShapeDtypeStruct((E, 2 * H), jnp.float32),
        mesh=_sc_mesh(),
        scratch_types=[
            pltpu.VMEM((EPW,), jnp.int32),
            pltpu.VMEM((EPW,), jnp.int32),
            pltpu.VMEM((CH, 2 * H), jnp.float32),
            pltpu.VMEM((CH, 2 * H), jnp.float32),
            pltpu.SemaphoreType.DMA,
            pltpu.SemaphoreType.DMA,
            pltpu.SemaphoreType.DMA,
            pltpu.SemaphoreType.DMA,
            pltpu.SemaphoreType.DMA,
            pltpu.SemaphoreType.DMA,
        ],
    )


def _sc_gather(ap, bv, row, col):
    return _sc_gather_kernel()(ap, bv, row, col)


_DR = 16
_NDCH = N // _DR
CHS = 80
NCHS = EPW // CHS
_NB = 4


def _sc_scatter_body(ea_hbm, row_hbm, out_hbm, idxs, ebs, dbuf, shared,
                     sis, ses, sss):
    c = lax.axis_index("c")
    s = lax.axis_index("s")
    zero = jnp.zeros((16,), jnp.float32)

    def zfill(i, carry):
        for k in range(2 * H // 16):
            dbuf[i, pl.ds(k * 16, 16)] = zero
        return carry

    lax.fori_loop(0, _DR, zfill, 0)
    nd = 39 + jnp.where(s == 0, 1, 0)

    def zbody(i, carry):
        pltpu.sync_copy(dbuf, shared.at[pl.ds((s + i * NS) * _DR, _DR)])
        return carry

    lax.fori_loop(0, nd, zbody, 0)
    plsc.subcore_barrier()

    base0 = (c * NS + s) * EPW

    def do_group(gbase, nb):
        di = []
        de = []
        for b in range(nb):
            o = gbase + b * CHS
            di.append(pltpu.async_copy(row_hbm.at[pl.ds(o, CHS)],
                                       idxs[b].at[0], sis[b]))
            de.append(pltpu.async_copy(ea_hbm.at[pl.ds(o, CHS)],
                                       ebs[b], ses[b]))
        fs = []
        for b in range(nb):
            di[b].wait()
            de[b].wait()
            fs.append(pltpu.async_copy(ebs[b], shared.at[idxs[b].at[0]],
                                       sss[b], add=True))
        for f in fs:
            f.wait()

    def body(i, carry):
        do_group(base0 + i * (_NB * CHS), _NB)
        return carry

    lax.fori_loop(0, NCHS // _NB, body, 0)
    do_group(base0 + (NCHS // _NB) * _NB * CHS, NCHS % _NB)
    plsc.subcore_barrier()

    def dbody(i, carry):
        rbase = (s + i * NS) * _DR
        pltpu.sync_copy(shared.at[pl.ds(rbase, _DR)], dbuf)
        pltpu.sync_copy(dbuf, out_hbm.at[pl.ds(c * N + rbase, _DR)])
        return carry

    lax.fori_loop(0, nd, dbody, 0)


@functools.cache
def _sc_scatter_kernel():
    return pl.kernel(
        _sc_scatter_body,
        out_type=jax.ShapeDtypeStruct((NC * N, 2 * H), jnp.float32),
        mesh=_sc_mesh(),
        scratch_types=[
            [pltpu.VMEM((1, CHS), jnp.int32) for _ in range(_NB)],
            [pltpu.VMEM((CHS, 2 * H), jnp.float32) for _ in range(_NB)],
            pltpu.VMEM((_DR, 2 * H), jnp.float32),
            pltpu.VMEM_SHARED((N, 2 * H), jnp.float32),
            [pltpu.SemaphoreType.DMA for _ in range(_NB)],
            [pltpu.SemaphoreType.DMA for _ in range(_NB)],
            [pltpu.SemaphoreType.DMA for _ in range(_NB)],
        ],
    )


def _sc_scatter(ea, row):
    return _sc_scatter_kernel()(ea, row)


def _full(shape):
    nd = len(shape)
    return pl.BlockSpec(shape, lambda i, _nd=nd: (0,) * _nd)


def _rows(blk, width):
    return pl.BlockSpec((blk, width), lambda i: (i, 0))


def _mm(a, b):
    return jnp.dot(a, b, precision=lax.Precision.HIGHEST)


def _graph_body(cond, ceW1, ceb1, ceW2, ceb2, Wv, bv, Wo, bo,
                Wu, peb1, Watt, pnb1, u_out, att_out):
    u = _mm(jnp.maximum(_mm(cond[...], ceW1[...]) + ceb1[...], 0.0),
            ceW2[...]) + ceb2[...]
    att = _mm(_mm(u, Wv[...]) + bv[...], Wo[...]) + bo[...]
    for l in range(L):
        u_out[l] = _mm(u, Wu[l]) + peb1[l]
        att_out[l] = _mm(att, Watt[l]) + pnb1[l]


def _enc_node_body(x, bat, neW1, neb1, neW2, neb2, Wr, Wc, U0,
                   h_out, a_out, b_out):
    h = _mm(jnp.maximum(_mm(x[...], neW1[...]) + neb1[...], 0.0),
            neW2[...]) + neb2[...]
    oh = (bat[...] == lax.broadcasted_iota(jnp.int32, (NBLK, B), 1)
          ).astype(jnp.float32)
    z = jnp.zeros((NBLK, H), jnp.float32)
    h_out[...] = h
    a_out[...] = jnp.concatenate([_mm(h, Wr[...]) + _mm(oh, U0[...]), z],
                                 axis=1)
    b_out[...] = jnp.concatenate([z, _mm(h, Wc[...])], axis=1)


def _enc_edge_body(eattr, W1, b1, W2, b2, out):
    ea = _mm(jnp.maximum(_mm(eattr[...], W1[...]) + b1[...], 0.0),
             W2[...]) + b2[...]
    out[...] = jnp.concatenate([ea, jnp.zeros((EBLK, H), jnp.float32)], axis=1)


def _edge_mlp_body(g, ea, We, W2, b2, out):
    gv = g[...]
    hid = jnp.maximum(gv[:, :H] + gv[:, H:] + _mm(ea[:, :H], We[...]), 0.0)
    eo = _mm(hid, W2[...]) + b2[...]
    out[...] = jnp.concatenate([eo, jnp.zeros((EBLK, H), jnp.float32)], axis=1)


def _node_body(h, agg0, agg1, bat, Wh, Wagg, ATTl, W2, b2, Wr, Wc, Un,
               h_out, a_out, b_out):
    hv = h[...]
    oh = (bat[...] == lax.broadcasted_iota(jnp.int32, (NBLK, B), 1)
          ).astype(jnp.float32)
    pre = (_mm(hv, Wh[...]) + _mm(agg0[:, :H] + agg1[:, :H], Wagg[...])
           + _mm(oh, ATTl[...]))
    hn = _mm(jnp.maximum(pre, 0.0), W2[...]) + b2[...] + hv
    z = jnp.zeros((NBLK, H), jnp.float32)
    h_out[...] = hn
    a_out[...] = jnp.concatenate([_mm(hn, Wr[...]) + _mm(oh, Un[...]), z],
                                 axis=1)
    b_out[...] = jnp.concatenate([z, _mm(hn, Wc[...])], axis=1)


def _dec_body(h, W1, b1, W2, b2, out):
    out[...] = _mm(jnp.maximum(_mm(h[...], W1[...]) + b1[...], 0.0),
                   W2[...]) + b2[...]


def kernel(x, edge_index, edge_attr, conditions, batch,
           ne_W1, ne_b1, ne_W2, ne_b2, ee_W1, ee_b1, ee_W2, ee_b2,
           ce_W1, ce_b1, ce_W2, ce_b2,
           a_Wq, a_bq, a_Wk, a_bk, a_Wv, a_bv, a_Wo, a_bo,
           pe_W1, pe_b1, pe_W2, pe_b2, pn_W1, pn_b1, pn_W2, pn_b2,
           dec_W1, dec_b1, dec_W2, dec_b2):
    f32 = jnp.float32
    row = edge_index[0]
    col = edge_index[1]
    bat2 = batch.reshape(N, 1)

    Wr = pe_W1[:, 0:H, :]
    Wc = pe_W1[:, H:2 * H, :]
    We = pe_W1[:, 2 * H:3 * H, :]
    Wu = pe_W1[:, 3 * H:4 * H, :]
    Wh = pn_W1[:, 0:H, :]
    Wagg = pn_W1[:, H:2 * H, :]
    Watt = pn_W1[:, 2 * H:3 * H, :]

    r1 = lambda b: b.reshape(1, -1)
    r2 = lambda b: b.reshape(L, 1, -1)

    U, ATT = pl.pallas_call(
        _graph_body,
        out_shape=[jax.ShapeDtypeStruct((L, B, H), f32),
                   jax.ShapeDtypeStruct((L, B, H), f32)],
    )(conditions, ce_W1, r1(ce_b1), ce_W2, r1(ce_b2),
      a_Wv, r1(a_bv), a_Wo, r1(a_bo), Wu, r2(pe_b1), Watt, r2(pn_b1))

    gn = N // NBLK
    h, Ap, Bv = pl.pallas_call(
        _enc_node_body,
        grid=(gn,),
        in_specs=[_rows(NBLK, 128), _rows(NBLK, 1),
                  _full((128, H)), _full((1, H)), _full((H, H)), _full((1, H)),
                  _full((H, H)), _full((H, H)), _full((B, H))],
        out_specs=[_rows(NBLK, H), _rows(NBLK, 2 * H), _rows(NBLK, 2 * H)],
        out_shape=[jax.ShapeDtypeStruct((N, H), f32),
                   jax.ShapeDtypeStruct((N, 2 * H), f32),
                   jax.ShapeDtypeStruct((N, 2 * H), f32)],
    )(x, bat2, ne_W1, r1(ne_b1), ne_W2, r1(ne_b2), Wr[0], Wc[0], U[0])

    ge = E // EBLK
    ea = pl.pallas_call(
        _enc_edge_body,
        grid=(ge,),
        in_specs=[_rows(EBLK, 16), _full((16, H)), _full((1, H)),
                  _full((H, H)), _full((1, H))],
        out_specs=_rows(EBLK, 2 * H),
        out_shape=jax.ShapeDtypeStruct((E, 2 * H), f32),
    )(edge_attr, ee_W1, r1(ee_b1), ee_W2, r1(ee_b2))

    for l in range(L):
        G = _sc_gather(Ap, Bv, row, col)
        ea = pl.pallas_call(
            _edge_mlp_body,
            grid=(ge,),
            in_specs=[_rows(EBLK, 2 * H), _rows(EBLK, 2 * H), _full((H, H)),
                      _full((H, H)), _full((1, H))],
            out_specs=_rows(EBLK, 2 * H),
            out_shape=jax.ShapeDtypeStruct((E, 2 * H), f32),
        )(G, ea, We[l], pe_W2[l], r1(pe_b2[l]))

        aggs = _sc_scatter(ea, row)
        ln = (l + 1) % L
        h, Ap, Bv = pl.pallas_call(
            _node_body,
            grid=(gn,),
            in_specs=[_rows(NBLK, H), _rows(NBLK, 2 * H), _rows(NBLK, 2 * H),
                      _rows(NBLK, 1)] +
                     [_full((H, H))] * 2 + [_full((B, H)), _full((H, H)),
                      _full((1, H)), _full((H, H)), _full((H, H)),
                      _full((B, H))],
            out_specs=[_rows(NBLK, H), _rows(NBLK, 2 * H), _rows(NBLK, 2 * H)],
            out_shape=[jax.ShapeDtypeStruct((N, H), f32),
                       jax.ShapeDtypeStruct((N, 2 * H), f32),
                       jax.ShapeDtypeStruct((N, 2 * H), f32)],
        )(h, aggs[:N], aggs[N:], bat2, Wh[l], Wagg[l], ATT[l],
          pn_W2[l], r1(pn_b2[l]), Wr[ln], Wc[ln], U[ln])

    out = pl.pallas_call(
        _dec_body,
        grid=(gn,),
        in_specs=[_rows(NBLK, H), _full((H, H)), _full((1, H)),
                  _full((H, 128)), _full((1, 128))],
        out_specs=_rows(NBLK, 128),
        out_shape=jax.ShapeDtypeStruct((N, 128), f32),
    )(h, dec_W1, r1(dec_b1), dec_W2, r1(dec_b2))
    return out
